# baseline (speedup 1.0000x reference)
import functools

import jax
import jax.numpy as jnp
import numpy as np
from jax import lax
from jax.experimental import pallas as pl
from jax.experimental.pallas import tpu as pltpu
from jax.experimental.pallas import tpu_sc as plsc

_NC = 2
_NS = 16
_NW = _NC * _NS

_BLK = 128
_L = 16
_NBUF = 4


@functools.lru_cache(maxsize=None)
def _build_call(n_t: int, n_b: int, hidden: int, vocab: int, scale: float):
    n_tasks = n_t * (n_b // _BLK)
    per_w = n_tasks // _NW
    nb_blk = n_b // _BLK
    groups = _BLK // _L

    mesh = plsc.VectorSubcoreMesh(core_axis_name="c", subcore_axis_name="s")

    @functools.partial(
        pl.kernel,
        mesh=mesh,
        out_type=jax.ShapeDtypeStruct((n_t, hidden, n_b), jnp.float32),
        scratch_types=[
            pltpu.VMEM((per_w, _BLK), jnp.int32),
            pltpu.VMEM((_NBUF, _BLK), jnp.int32),
            pltpu.VMEM((_NBUF, _BLK, 2 * hidden), jnp.float32),
            pltpu.VMEM((_NBUF, hidden, _BLK), jnp.float32),
            pltpu.SemaphoreType.DMA,
            pltpu.SemaphoreType.DMA,
        ],
        compiler_params=pltpu.CompilerParams(needs_layout_passes=False),
    )
    def emb(idx_hbm, wt_hbm, out_hbm, idxall, idxg, gbuf, obuf, gsem, osem):
        wid = lax.axis_index("s") * _NC + lax.axis_index("c")
        task0 = wid * per_w

        pltpu.sync_copy(idx_hbm.at[pl.ds(task0, per_w)], idxall)

        def prep_and_fire(kk, b):
            for j in range(groups):
                sl = pl.ds(j * _L, _L)
                idxg[b, sl] = lax.shift_right_logical(idxall[kk, sl], 1)
            pltpu.async_copy(wt_hbm.at[idxg.at[b]], gbuf.at[b], gsem)

        def wait_gather(b):
            pltpu.make_async_copy(wt_hbm.at[idxg.at[b]], gbuf.at[b], gsem).wait()

        def out_slice(kk):
            gk = task0 + kk
            t = gk // nb_blk
            b0 = (gk % nb_blk) * _BLK
            return out_hbm.at[t, :, pl.ds(b0, _BLK)]

        for b in range(_NBUF):
            prep_and_fire(b, b)

        iota = lax.iota(jnp.int32, _L)

        def round_body(g, carry):
            for b in range(_NBUF):
                kk = g * _NBUF + b
                wait_gather(b)

                @pl.when(g > 0)
                def _():
                    pltpu.make_async_copy(
                        obuf.at[b], out_slice(kk - _NBUF), osem
                    ).wait()

                rows = []
                cols = []
                for j in range(groups):
                    sl = pl.ds(j * _L, _L)
                    rows.append(iota + (j * _L))
                    raw = idxall[kk, sl]
                    cols.append(
                        lax.shift_left(lax.bitwise_and(raw, 1), 6)
                    )

                def h_body(h, c):
                    for j in range(groups):
                        v = plsc.load_gather(gbuf.at[b], [rows[j], cols[j] + h])
                        obuf[b, h, pl.ds(j * _L, _L)] = v * scale
                    return c

                lax.fori_loop(0, hidden, h_body, 0, unroll=2)

                pltpu.async_copy(obuf.at[b], out_slice(kk), osem)

                @pl.when(kk + _NBUF < per_w)
                def _():
                    prep_and_fire(kk + _NBUF, b)
            return carry

        lax.fori_loop(0, per_w // _NBUF, round_body, 0)

        for b in range(_NBUF):
            kk = per_w - _NBUF + b
            pltpu.make_async_copy(obuf.at[b], out_slice(kk), osem).wait()

    return emb


def kernel(x, weights):
    n_b, n_t = x.shape
    vocab, hidden = weights.shape
    scale = float(np.float32(np.sqrt(np.float32(hidden))))
    idx2d = jnp.transpose(x).reshape(n_tasks_rows := n_b * n_t // _BLK, _BLK)
    idx2d = idx2d.astype(jnp.int32)
    wt = weights.reshape(vocab // 2, 2 * hidden)
    out = _build_call(n_t, n_b, hidden, vocab, scale)(idx2d, wt)
    return jnp.transpose(out, (2, 0, 1))


# speedup vs baseline: 1.5332x; 1.5332x over previous
import functools

import jax
import jax.numpy as jnp
import numpy as np
from jax import lax
from jax.experimental import pallas as pl
from jax.experimental.pallas import tpu as pltpu
from jax.experimental.pallas import tpu_sc as plsc

_NC = 2
_NS = 16
_NW = _NC * _NS

_BLK = 128
_L = 16
_NBUF = 2


@functools.lru_cache(maxsize=None)
def _build_call(n_t: int, n_b: int, hidden: int, vocab: int, scale: float):
    n_tasks = n_t * (n_b // _BLK)
    per_w = n_tasks // _NW
    nb_blk = n_b // _BLK
    groups = _BLK // _L

    mesh = plsc.VectorSubcoreMesh(core_axis_name="c", subcore_axis_name="s")

    @functools.partial(
        pl.kernel,
        mesh=mesh,
        out_type=jax.ShapeDtypeStruct((n_t, hidden, n_b), jnp.float32),
        scratch_types=[
            pltpu.VMEM((per_w, _BLK), jnp.int32),
            pltpu.VMEM((_NBUF, _BLK), jnp.int32),
            pltpu.VMEM((_NBUF, _BLK, 2 * hidden + 1), jnp.float32),
            pltpu.VMEM((_NBUF, hidden, _BLK), jnp.float32),
            pltpu.SemaphoreType.DMA,
            pltpu.SemaphoreType.DMA,
        ],
        compiler_params=pltpu.CompilerParams(needs_layout_passes=False),
    )
    def emb(idx_hbm, wt_hbm, out_hbm, idxall, idxg, gbuf, obuf, gsem, osem):
        wid = lax.axis_index("s") * _NC + lax.axis_index("c")
        task0 = wid * per_w

        pltpu.sync_copy(idx_hbm.at[pl.ds(task0, per_w)], idxall)

        def prep_and_fire(kk, b):
            for j in range(groups):
                sl = pl.ds(j * _L, _L)
                idxg[b, sl] = lax.shift_right_logical(idxall[kk, sl], 1)
            pltpu.async_copy(
                wt_hbm.at[idxg.at[b]],
                gbuf.at[b, :, pl.ds(0, 2 * hidden)],
                gsem,
            )

        def wait_gather(b):
            pltpu.make_async_copy(
                wt_hbm.at[idxg.at[b]],
                gbuf.at[b, :, pl.ds(0, 2 * hidden)],
                gsem,
            ).wait()

        def out_slice(kk):
            gk = task0 + kk
            t = gk // nb_blk
            b0 = (gk % nb_blk) * _BLK
            return out_hbm.at[t, :, pl.ds(b0, _BLK)]

        for b in range(_NBUF):
            prep_and_fire(b, b)

        iota = lax.iota(jnp.int32, _L)

        def round_body(g, carry):
            for b in range(_NBUF):
                kk = g * _NBUF + b
                wait_gather(b)

                @pl.when(g > 0)
                def _():
                    pltpu.make_async_copy(
                        obuf.at[b], out_slice(kk - _NBUF), osem
                    ).wait()

                rows = []
                cols = []
                for j in range(groups):
                    sl = pl.ds(j * _L, _L)
                    parity = lax.shift_left(
                        lax.bitwise_and(idxall[kk, sl], 1), 6
                    )
                    rows.append(iota + (j * _L))
                    cols.append(parity)

                @plsc.parallel_loop(0, hidden, unroll=8)
                def _(h):
                    for j in range(groups):
                        v = plsc.load_gather(
                            gbuf.at[b], [rows[j], cols[j] + h]
                        )
                        obuf[b, h, pl.ds(j * _L, _L)] = v * scale

                pltpu.async_copy(obuf.at[b], out_slice(kk), osem)

                @pl.when(kk + _NBUF < per_w)
                def _():
                    prep_and_fire(kk + _NBUF, b)
            return carry

        lax.fori_loop(0, per_w // _NBUF, round_body, 0)

        for b in range(_NBUF):
            kk = per_w - _NBUF + b
            pltpu.make_async_copy(obuf.at[b], out_slice(kk), osem).wait()

    return emb


def kernel(x, weights):
    n_b, n_t = x.shape
    vocab, hidden = weights.shape
    scale = float(np.float32(np.sqrt(np.float32(hidden))))
    idx2d = jnp.transpose(x).reshape(n_tasks_rows := n_b * n_t // _BLK, _BLK)
    idx2d = idx2d.astype(jnp.int32)
    wt = weights.reshape(vocab // 2, 2 * hidden)
    out = _build_call(n_t, n_b, hidden, vocab, scale)(idx2d, wt)
    return jnp.transpose(out, (2, 0, 1))
